# baseline (speedup 1.0000x reference)
import jax
import jax.numpy as jnp
from jax import lax
from jax.experimental import pallas as pl
from jax.experimental.pallas import tpu as pltpu
from jax.experimental.pallas import tpu_sc as plsc

N = 10000
E = 320000
D = 128
NEG_SLOPE = 0.2
ACT_SLOPE = 0.01

NC = 2
NS = 16
NW = NC * NS
EW = 10240
EPAD = NW * EW
NPAD = 10240
RB = N // 5

CH = 128
NCHUNK = EW // CH
GROUPS = CH // 16

P2CH = 64
SB = 2048
NSB = EW // SB
SCHUNK = SB // P2CH
GROUPS2 = P2CH // 16
STRIPE = NPAD // NS

_IPRET = False

_mesh = plsc.VectorSubcoreMesh(core_axis_name="c", subcore_axis_name="s",
                               num_cores=NC, num_subcores=NS)


def _lrelu(v, slope):
    return jnp.where(v >= 0, v, slope * v)


def _pexp(t):
    u = 1.0 / 120.0 + t * (1.0 / 720.0)
    u = 1.0 / 24.0 + t * u
    u = 1.0 / 6.0 + t * u
    u = 0.5 + t * u
    u = 1.0 + t * u
    return 1.0 + t * u


def _tc_proj_body(x_ref, ws_ref, bs_ref, wd_ref, bd_ref, fs_ref, fd_ref):
    x = x_ref[...]
    fs_ref[...] = jnp.dot(x, ws_ref[...], preferred_element_type=jnp.float32) + bs_ref[...]
    fd_ref[...] = jnp.dot(x, wd_ref[...], preferred_element_type=jnp.float32) + bd_ref[...]


def _tc_proj(x, ws, bs, wd, bd):
    row = pl.BlockSpec((RB, D), lambda i: (i, 0))
    full = pl.BlockSpec((D, D), lambda i: (0, 0))
    vec = pl.BlockSpec((1, D), lambda i: (0, 0))
    return pl.pallas_call(
        _tc_proj_body,
        grid=(5,),
        in_specs=[row, full, vec, full, vec],
        out_specs=[row, row],
        out_shape=[jax.ShapeDtypeStruct((N, D), jnp.float32)] * 2,
        interpret=_IPRET,
    )(x, ws, bs.reshape(1, D), wd, bd.reshape(1, D))


def _tc_rsum_body(s_ref, o_ref):
    o_ref[...] = jnp.sum(s_ref[...], axis=0, keepdims=True)


def _tc_rsum(ssum):
    return pl.pallas_call(
        _tc_rsum_body,
        grid=(1,),
        in_specs=[pl.BlockSpec((NW, NPAD), lambda i: (0, 0))],
        out_specs=pl.BlockSpec((1, NPAD), lambda i: (0, 0)),
        out_shape=jax.ShapeDtypeStruct((1, NPAD), jnp.float32),
        interpret=_IPRET,
    )(ssum)


def _tc_mid_body(x_ref, g0_ref, g1_ref, w1_ref, b1_ref, ws_ref, bs_ref,
                 wd_ref, bd_ref, x1_ref, fs_ref, fd_ref):
    h = x_ref[...] + g0_ref[...] + g1_ref[...]
    x1 = _lrelu(jnp.dot(h, w1_ref[...], preferred_element_type=jnp.float32) + b1_ref[...],
                ACT_SLOPE)
    x1_ref[...] = x1
    fs_ref[...] = jnp.dot(x1, ws_ref[...], preferred_element_type=jnp.float32) + bs_ref[...]
    fd_ref[...] = jnp.dot(x1, wd_ref[...], preferred_element_type=jnp.float32) + bd_ref[...]


def _tc_mid(x, g0, g1, w1, b1, ws, bs, wd, bd):
    row = pl.BlockSpec((RB, D), lambda i: (i, 0))
    full = pl.BlockSpec((D, D), lambda i: (0, 0))
    vec = pl.BlockSpec((1, D), lambda i: (0, 0))
    return pl.pallas_call(
        _tc_mid_body,
        grid=(5,),
        in_specs=[row, row, row, full, vec, full, vec, full, vec],
        out_specs=[row, row, row],
        out_shape=[jax.ShapeDtypeStruct((N, D), jnp.float32)] * 3,
        interpret=_IPRET,
    )(x, g0, g1, w1, b1.reshape(1, D), ws, bs.reshape(1, D), wd, bd.reshape(1, D))


def _tc_fin_body(x_ref, g0_ref, g1_ref, w1_ref, b1_ref, x2_ref, gt_ref):
    gt = g0_ref[...] + g1_ref[...]
    gt_ref[...] = gt
    h = x_ref[...] + gt
    x2_ref[...] = _lrelu(jnp.dot(h, w1_ref[...], preferred_element_type=jnp.float32) + b1_ref[...],
                         ACT_SLOPE)


def _tc_fin(x, g0, g1, w1, b1):
    row = pl.BlockSpec((RB, D), lambda i: (i, 0))
    full = pl.BlockSpec((D, D), lambda i: (0, 0))
    vec = pl.BlockSpec((1, D), lambda i: (0, 0))
    return pl.pallas_call(
        _tc_fin_body,
        grid=(5,),
        in_specs=[row, row, row, full, vec],
        out_specs=[row, row],
        out_shape=[jax.ShapeDtypeStruct((N, D), jnp.float32)] * 2,
        interpret=_IPRET,
    )(x, g0, g1, w1, b1.reshape(1, D))


def _sc_pass1_body(fs_hbm, fd_hbm, src_hbm, dst_hbm, attn_hbm,
                   ex_hbm, ssum_hbm,
                   srcv, dstv, exv, attnv, ssumv, tbuf,
                   fsb0, fsb1, fdb0, fdb1, sem0, sem1):
    cid = lax.axis_index("c")
    sid = lax.axis_index("s")
    wid = sid * NC + cid
    base = wid * EW
    fsb = (fsb0, fsb1)
    fdb = (fdb0, fdb1)
    sem = (sem0, sem1)

    pltpu.sync_copy(attn_hbm, attnv)
    pltpu.sync_copy(src_hbm.at[pl.ds(base, EW)], srcv)
    pltpu.sync_copy(dst_hbm.at[pl.ds(base, EW)], dstv)

    def zbody(i, _):
        ssumv[pl.ds(i * 16, 16)] = jnp.zeros((16,), jnp.float32)
        return 0
    lax.fori_loop(0, NPAD // 16, zbody, 0, unroll=8)

    iota = lax.iota(jnp.int32, 16)
    attn_k = [attnv[pl.ds(k * 16, 16)] for k in range(D // 16)]

    def _issue(c, b):
        pltpu.async_copy(fs_hbm.at[srcv.at[pl.ds(c * CH, CH)]], fsb[b], sem[b])
        pltpu.async_copy(fd_hbm.at[dstv.at[pl.ds(c * CH, CH)]], fdb[b], sem[b])

    def _wait(c, b):
        pltpu.make_async_copy(fs_hbm.at[srcv.at[pl.ds(c * CH, CH)]], fsb[b], sem[b]).wait()
        pltpu.make_async_copy(fd_hbm.at[dstv.at[pl.ds(c * CH, CH)]], fdb[b], sem[b]).wait()

    _issue(0, 0)
    _issue(1, 1)

    def chunk2(cc, _):
        for b in range(2):
            c = cc * 2 + b
            _wait(c, b)

            def group(g, _):
                eb = c * CH + g * 16
                for j4 in range(0, 16, 4):
                    accs = [jnp.zeros((16,), jnp.float32) for _ in range(4)]
                    for k in range(D // 16):
                        for u in range(4):
                            e = g * 16 + j4 + u
                            v = fsb[b][e, pl.ds(k * 16, 16)] + fdb[b][e, pl.ds(k * 16, 16)]
                            accs[u] = accs[u] + attn_k[k] * _lrelu(v, NEG_SLOPE)
                    for u in range(4):
                        plsc.store_scatter(tbuf, [iota * 16 + j4 + u], accs[u])
                e16 = tbuf[pl.ds(0, 16)]
                for i in range(1, 16):
                    e16 = e16 + tbuf[pl.ds(i * 16, 16)]
                ex16 = jnp.exp(e16)
                exv[pl.ds(eb, 16)] = ex16
                dst16 = dstv[pl.ds(eb, 16)]
                plsc.addupdate_scatter(ssumv, [dst16], ex16)
                return 0

            lax.fori_loop(0, GROUPS, group, 0)

            @pl.when(c < NCHUNK - 2)
            def _():
                _issue(c + 2, b)
        return 0

    lax.fori_loop(0, NCHUNK // 2, chunk2, 0)
    pltpu.sync_copy(exv, ex_hbm.at[pl.ds(base, EW)])
    pltpu.sync_copy(ssumv, ssum_hbm.at[wid])


def _sc_pass1(fs, fd, srcp, dstp, attn):
    f = pl.kernel(
        _sc_pass1_body,
        out_type=[jax.ShapeDtypeStruct((EPAD,), jnp.float32),
                  jax.ShapeDtypeStruct((NW, NPAD), jnp.float32)],
        mesh=_mesh,
        scratch_types=[
            pltpu.VMEM((EW,), jnp.int32),
            pltpu.VMEM((EW,), jnp.int32),
            pltpu.VMEM((EW,), jnp.float32),
            pltpu.VMEM((D,), jnp.float32),
            pltpu.VMEM((NPAD,), jnp.float32),
            pltpu.VMEM((256,), jnp.float32),
            pltpu.VMEM((CH, D), jnp.float32),
            pltpu.VMEM((CH, D), jnp.float32),
            pltpu.VMEM((CH, D), jnp.float32),
            pltpu.VMEM((CH, D), jnp.float32),
            pltpu.SemaphoreType.DMA,
            pltpu.SemaphoreType.DMA,
        ],
        compiler_params=pltpu.CompilerParams(needs_layout_passes=False),
        interpret=_IPRET,
    )
    return f(fs, fd, srcp, dstp, attn)


def _sc_pass2_body(fs_hbm, src_hbm, dst_hbm, r_hbm, ex_hbm,
                   ssum_hbm, rel_hbm,
                   a_hbm, g_hbm,
                   srcv, dstv, rv, exv, av, relv, ssumv, tbuf, msgb,
                   fsb0, fsb1, sem0, sem1, gsh):
    cid = lax.axis_index("c")
    sid = lax.axis_index("s")
    wid = sid * NC + cid
    base = wid * EW
    fsb = (fsb0, fsb1)
    sem = (sem0, sem1)

    pltpu.sync_copy(rel_hbm, relv)
    pltpu.sync_copy(ssum_hbm, ssumv)

    def zb(i, _):
        r = i // 8
        k = i % 8
        msgb[r, pl.ds(k * 16, 16)] = jnp.zeros((16,), jnp.float32)
        return 0
    lax.fori_loop(0, P2CH * 8, zb, 0, unroll=8)
    for j in range(STRIPE // P2CH):
        pltpu.sync_copy(msgb, gsh.at[pl.ds(sid * STRIPE + j * P2CH, P2CH)])
    plsc.subcore_barrier()

    iota = lax.iota(jnp.int32, 16)

    def _issue(c, b):
        pltpu.async_copy(fs_hbm.at[srcv.at[pl.ds(c * P2CH, P2CH)]], fsb[b], sem[b])

    def _wait(c, b):
        pltpu.make_async_copy(fs_hbm.at[srcv.at[pl.ds(c * P2CH, P2CH)]], fsb[b], sem[b]).wait()

    def superchunk(s, _):
        sbase = base + s * SB
        pltpu.sync_copy(src_hbm.at[pl.ds(sbase, SB)], srcv)
        pltpu.sync_copy(dst_hbm.at[pl.ds(sbase, SB)], dstv)
        pltpu.sync_copy(r_hbm.at[pl.ds(sbase, SB)], rv)
        pltpu.sync_copy(ex_hbm.at[pl.ds(sbase, SB)], exv)
        _issue(0, 0)
        _issue(1, 1)

        def chunk2(cc, _):
            for b in range(2):
                c = cc * 2 + b
                _wait(c, b)

                def group(g, _):
                    off = c * P2CH + g * 16
                    dst16 = dstv[pl.ds(off, 16)]
                    ex16 = exv[pl.ds(off, 16)]
                    sv = plsc.load_gather(ssumv, [dst16])
                    a16 = ex16 / (sv + 1e-9)
                    av[pl.ds(off, 16)] = a16
                    r16 = rv[pl.ds(off, 16)]
                    res = [r16[j] for j in range(16)]
                    for j4 in range(0, 16, 4):
                        saccs = [jnp.zeros((16,), jnp.float32) for _ in range(4)]
                        for k in range(D // 16):
                            for u in range(4):
                                e = g * 16 + j4 + u
                                t = (fsb[b][e, pl.ds(k * 16, 16)]
                                     * relv[res[j4 + u], pl.ds(k * 16, 16)]
                                     * a16[j4 + u])
                                p = _pexp(t)
                                msgb[e, pl.ds(k * 16, 16)] = p
                                saccs[u] = saccs[u] + p
                        for u in range(4):
                            plsc.store_scatter(tbuf, [iota * 16 + j4 + u], saccs[u])
                    s16 = tbuf[pl.ds(0, 16)]
                    for i in range(1, 16):
                        s16 = s16 + tbuf[pl.ds(i * 16, 16)]
                    inv16 = 1.0 / s16
                    for k in range(D // 16):
                        for j in range(16):
                            e = g * 16 + j
                            msgb[e, pl.ds(k * 16, 16)] = msgb[e, pl.ds(k * 16, 16)] * inv16[j]
                    pltpu.sync_copy(msgb.at[pl.ds(g * 16, 16)], gsh.at[dst16], add=True)
                    return 0

                lax.fori_loop(0, GROUPS2, group, 0)

                @pl.when(c < SCHUNK - 2)
                def _():
                    _issue(c + 2, b)
            return 0

        lax.fori_loop(0, SCHUNK // 2, chunk2, 0)
        pltpu.sync_copy(av, a_hbm.at[pl.ds(sbase, SB)])
        return 0

    lax.fori_loop(0, NSB, superchunk, 0)
    plsc.subcore_barrier()
    for j in range(STRIPE // P2CH):
        r0 = sid * STRIPE + j * P2CH
        pltpu.sync_copy(gsh.at[pl.ds(r0, P2CH)],
                        g_hbm.at[pl.ds(cid * NPAD + r0, P2CH)])


def _sc_pass2(fs, srcp, dstp, rp, ex, ssum_tot, rel):
    f = pl.kernel(
        _sc_pass2_body,
        out_type=[jax.ShapeDtypeStruct((EPAD,), jnp.float32),
                  jax.ShapeDtypeStruct((NC * NPAD, D), jnp.float32)],
        mesh=_mesh,
        scratch_types=[
            pltpu.VMEM((SB,), jnp.int32),
            pltpu.VMEM((SB,), jnp.int32),
            pltpu.VMEM((SB,), jnp.int32),
            pltpu.VMEM((SB,), jnp.float32),
            pltpu.VMEM((SB,), jnp.float32),
            pltpu.VMEM((16, D), jnp.float32),
            pltpu.VMEM((NPAD,), jnp.float32),
            pltpu.VMEM((256,), jnp.float32),
            pltpu.VMEM((P2CH, D), jnp.float32),
            pltpu.VMEM((P2CH, D), jnp.float32),
            pltpu.VMEM((P2CH, D), jnp.float32),
            pltpu.SemaphoreType.DMA,
            pltpu.SemaphoreType.DMA,
            pltpu.VMEM_SHARED((NPAD, D), jnp.float32),
        ],
        compiler_params=pltpu.CompilerParams(needs_layout_passes=False),
        interpret=_IPRET,
    )
    return f(fs, srcp, dstp, rp, ex, ssum_tot, rel)


def _layer(x, srcp, dstp, rp, rel, ws, bs, wd, bd, attn):
    fs, fd = _tc_proj(x, ws, bs, wd, bd)
    ex, ssum = _sc_pass1(fs, fd, srcp, dstp, attn)
    ssum_tot = _tc_rsum(ssum).reshape(NPAD)
    a, gflat = _sc_pass2(fs, srcp, dstp, rp, ex, ssum_tot, rel)
    g0 = gflat[:N]
    g1 = gflat[NPAD:NPAD + N]
    return a, g0, g1


@jax.jit
def kernel(x, edge_index, r_type, emb_rel, W1_0, b1_0, W2s_0, b2s_0, W2d_0,
           b2d_0, attn_0, W1_1, b1_1, W2s_1, b2s_1, W2d_1, b2d_1, attn_1):
    src = edge_index[0].astype(jnp.int32)
    dst = edge_index[1].astype(jnp.int32)
    rt = r_type.astype(jnp.int32)
    npad = EPAD - E
    srcp = jnp.concatenate([src, jnp.zeros((npad,), jnp.int32)])
    dstp = jnp.concatenate([dst, jnp.full((npad,), N, jnp.int32)])
    rp = jnp.concatenate([rt, jnp.zeros((npad,), jnp.int32)])

    a1, g0, g1 = _layer(x, srcp, dstp, rp, emb_rel,
                        W2s_0, b2s_0, W2d_0, b2d_0, attn_0.reshape(D))
    x1, fs2, fd2 = _tc_mid(x, g0, g1, W1_0, b1_0, W2s_1, b2s_1, W2d_1, b2d_1)

    ex2, ssum2 = _sc_pass1(fs2, fd2, srcp, dstp, attn_1.reshape(D))
    ssum2_tot = _tc_rsum(ssum2).reshape(NPAD)
    _, gflat2 = _sc_pass2(fs2, srcp, dstp, rp, ex2, ssum2_tot, emb_rel)
    x2, gnb = _tc_fin(x1, gflat2[:N], gflat2[NPAD:NPAD + N], W1_1, b1_1)

    emb = jnp.concatenate([x1, x2], axis=0)
    attentions = a1[:E].reshape(E, 1)
    return (x2, emb, gnb, attentions)


# speedup vs baseline: 3.0884x; 3.0884x over previous
import jax
import jax.numpy as jnp
from jax import lax
from jax.experimental import pallas as pl
from jax.experimental.pallas import tpu as pltpu
from jax.experimental.pallas import tpu_sc as plsc

N = 10000
E = 320000
D = 128
NEG_SLOPE = 0.2
ACT_SLOPE = 0.01

NC = 2
NS = 16
NW = NC * NS
EW = 10240
EPAD = NW * EW
NPAD = 10240
RB = N // 5

CH = 128
NCHUNK = EW // CH
GROUPS = CH // 16

P2CH = 64
SB = 2048
NSB = EW // SB
SCHUNK = SB // P2CH
GROUPS2 = P2CH // 16
STRIPE = NPAD // NS

_IPRET = False

_mesh = plsc.VectorSubcoreMesh(core_axis_name="c", subcore_axis_name="s",
                               num_cores=NC, num_subcores=NS)


def _lrelu(v, slope):
    return jnp.where(v >= 0, v, slope * v)


def _pexp4(t):
    t2 = t * t
    a = 1.0 + t
    b = 0.5 + t * (1.0 / 6.0)
    c = 1.0 / 24.0 + t * (1.0 / 120.0)
    return a + t2 * (b + t2 * c)


def _tc_proj_body(x_ref, ws_ref, bs_ref, wd_ref, bd_ref, fs_ref, fd_ref):
    x = x_ref[...]
    fs_ref[...] = jnp.dot(x, ws_ref[...], preferred_element_type=jnp.float32) + bs_ref[...]
    fd_ref[...] = jnp.dot(x, wd_ref[...], preferred_element_type=jnp.float32) + bd_ref[...]


def _tc_proj(x, ws, bs, wd, bd):
    row = pl.BlockSpec((RB, D), lambda i: (i, 0))
    full = pl.BlockSpec((D, D), lambda i: (0, 0))
    vec = pl.BlockSpec((1, D), lambda i: (0, 0))
    return pl.pallas_call(
        _tc_proj_body,
        grid=(5,),
        in_specs=[row, full, vec, full, vec],
        out_specs=[row, row],
        out_shape=[jax.ShapeDtypeStruct((N, D), jnp.float32)] * 2,
        interpret=_IPRET,
    )(x, ws, bs.reshape(1, D), wd, bd.reshape(1, D))


def _tc_rsum_body(s_ref, o_ref):
    o_ref[...] = jnp.sum(s_ref[...], axis=0, keepdims=True)


def _tc_rsum(ssum):
    return pl.pallas_call(
        _tc_rsum_body,
        grid=(1,),
        in_specs=[pl.BlockSpec((NW, NPAD), lambda i: (0, 0))],
        out_specs=pl.BlockSpec((1, NPAD), lambda i: (0, 0)),
        out_shape=jax.ShapeDtypeStruct((1, NPAD), jnp.float32),
        interpret=_IPRET,
    )(ssum)


def _tc_mid_body(x_ref, g0_ref, g1_ref, w1_ref, b1_ref, ws_ref, bs_ref,
                 wd_ref, bd_ref, x1_ref, fs_ref, fd_ref):
    h = x_ref[...] + g0_ref[...] + g1_ref[...]
    x1 = _lrelu(jnp.dot(h, w1_ref[...], preferred_element_type=jnp.float32) + b1_ref[...],
                ACT_SLOPE)
    x1_ref[...] = x1
    fs_ref[...] = jnp.dot(x1, ws_ref[...], preferred_element_type=jnp.float32) + bs_ref[...]
    fd_ref[...] = jnp.dot(x1, wd_ref[...], preferred_element_type=jnp.float32) + bd_ref[...]


def _tc_mid(x, g0, g1, w1, b1, ws, bs, wd, bd):
    row = pl.BlockSpec((RB, D), lambda i: (i, 0))
    full = pl.BlockSpec((D, D), lambda i: (0, 0))
    vec = pl.BlockSpec((1, D), lambda i: (0, 0))
    return pl.pallas_call(
        _tc_mid_body,
        grid=(5,),
        in_specs=[row, row, row, full, vec, full, vec, full, vec],
        out_specs=[row, row, row],
        out_shape=[jax.ShapeDtypeStruct((N, D), jnp.float32)] * 3,
        interpret=_IPRET,
    )(x, g0, g1, w1, b1.reshape(1, D), ws, bs.reshape(1, D), wd, bd.reshape(1, D))


def _tc_fin_body(x_ref, g0_ref, g1_ref, w1_ref, b1_ref, x2_ref, gt_ref):
    gt = g0_ref[...] + g1_ref[...]
    gt_ref[...] = gt
    h = x_ref[...] + gt
    x2_ref[...] = _lrelu(jnp.dot(h, w1_ref[...], preferred_element_type=jnp.float32) + b1_ref[...],
                         ACT_SLOPE)


def _tc_fin(x, g0, g1, w1, b1):
    row = pl.BlockSpec((RB, D), lambda i: (i, 0))
    full = pl.BlockSpec((D, D), lambda i: (0, 0))
    vec = pl.BlockSpec((1, D), lambda i: (0, 0))
    return pl.pallas_call(
        _tc_fin_body,
        grid=(5,),
        in_specs=[row, row, row, full, vec],
        out_specs=[row, row],
        out_shape=[jax.ShapeDtypeStruct((N, D), jnp.float32)] * 2,
        interpret=_IPRET,
    )(x, g0, g1, w1, b1.reshape(1, D))


def _sc_pass1_body(fs_hbm, fd_hbm, src_hbm, dst_hbm, attn_hbm,
                   ex_hbm, ssum_hbm,
                   srcv, dstv, exv, attnv, ssumv, tbuf,
                   fsb0, fsb1, fdb0, fdb1, sem0, sem1):
    cid = lax.axis_index("c")
    sid = lax.axis_index("s")
    wid = sid * NC + cid
    base = wid * EW
    fsb = (fsb0, fsb1)
    fdb = (fdb0, fdb1)
    sem = (sem0, sem1)

    pltpu.sync_copy(attn_hbm, attnv)
    pltpu.sync_copy(src_hbm.at[pl.ds(base, EW)], srcv)
    pltpu.sync_copy(dst_hbm.at[pl.ds(base, EW)], dstv)

    def zbody(i, _):
        ssumv[pl.ds(i * 16, 16)] = jnp.zeros((16,), jnp.float32)
        return 0
    lax.fori_loop(0, NPAD // 16, zbody, 0, unroll=8)

    iota = lax.iota(jnp.int32, 16)
    attn_k = [attnv[pl.ds(k * 16, 16)] for k in range(D // 16)]

    def _issue(c, b):
        pltpu.async_copy(fs_hbm.at[srcv.at[pl.ds(c * CH, CH)]], fsb[b], sem[b])
        pltpu.async_copy(fd_hbm.at[dstv.at[pl.ds(c * CH, CH)]], fdb[b], sem[b])

    def _wait(c, b):
        pltpu.make_async_copy(fs_hbm.at[srcv.at[pl.ds(c * CH, CH)]], fsb[b], sem[b]).wait()
        pltpu.make_async_copy(fd_hbm.at[dstv.at[pl.ds(c * CH, CH)]], fdb[b], sem[b]).wait()

    _issue(0, 0)
    _issue(1, 1)

    def chunk2(cc, _):
        for b in range(2):
            c = cc * 2 + b
            _wait(c, b)

            def group(g, _):
                eb = c * CH + g * 16
                for j4 in range(0, 16, 4):
                    es = [g * 16 + j4 + u for u in range(4)]
                    accs = [jnp.zeros((16,), jnp.float32) for _ in range(4)]
                    for k in range(D // 16):
                        ks = pl.ds(k * 16, 16)
                        vs = [fsb[b][e, ks] for e in es]
                        vd = [fdb[b][e, ks] for e in es]
                        v = [vs[u] + vd[u] for u in range(4)]
                        lr = [_lrelu(v[u], NEG_SLOPE) for u in range(4)]
                        accs = [accs[u] + attn_k[k] * lr[u] for u in range(4)]
                    for u in range(4):
                        plsc.store_scatter(tbuf, [iota * 16 + j4 + u], accs[u])
                e16 = tbuf[pl.ds(0, 16)]
                for i in range(1, 16):
                    e16 = e16 + tbuf[pl.ds(i * 16, 16)]
                ex16 = jnp.exp(e16)
                exv[pl.ds(eb, 16)] = ex16
                dst16 = dstv[pl.ds(eb, 16)]
                plsc.addupdate_scatter(ssumv, [dst16], ex16)
                return 0

            lax.fori_loop(0, GROUPS, group, 0)

            @pl.when(c < NCHUNK - 2)
            def _():
                _issue(c + 2, b)
        return 0

    lax.fori_loop(0, NCHUNK // 2, chunk2, 0)
    pltpu.sync_copy(exv, ex_hbm.at[pl.ds(base, EW)])
    pltpu.sync_copy(ssumv, ssum_hbm.at[wid])


def _sc_pass1(fs, fd, srcp, dstp, attn):
    f = pl.kernel(
        _sc_pass1_body,
        out_type=[jax.ShapeDtypeStruct((EPAD,), jnp.float32),
                  jax.ShapeDtypeStruct((NW, NPAD), jnp.float32)],
        mesh=_mesh,
        scratch_types=[
            pltpu.VMEM((EW,), jnp.int32),
            pltpu.VMEM((EW,), jnp.int32),
            pltpu.VMEM((EW,), jnp.float32),
            pltpu.VMEM((D,), jnp.float32),
            pltpu.VMEM((NPAD,), jnp.float32),
            pltpu.VMEM((256,), jnp.float32),
            pltpu.VMEM((CH, D), jnp.float32),
            pltpu.VMEM((CH, D), jnp.float32),
            pltpu.VMEM((CH, D), jnp.float32),
            pltpu.VMEM((CH, D), jnp.float32),
            pltpu.SemaphoreType.DMA,
            pltpu.SemaphoreType.DMA,
        ],
        compiler_params=pltpu.CompilerParams(needs_layout_passes=False),
        interpret=_IPRET,
    )
    return f(fs, fd, srcp, dstp, attn)


def _sc_pass2_body(fs_hbm, src_hbm, dst_hbm, r_hbm, ex_hbm,
                   ssum_hbm, rel_hbm,
                   a_hbm, g_hbm,
                   srcv, dstv, rv, exv, av, relv, ssumv, tbuf, msgb,
                   fsb0, fsb1, sem0, sem1, gsh):
    cid = lax.axis_index("c")
    sid = lax.axis_index("s")
    wid = sid * NC + cid
    base = wid * EW
    fsb = (fsb0, fsb1)
    sem = (sem0, sem1)

    pltpu.sync_copy(rel_hbm, relv)
    pltpu.sync_copy(ssum_hbm, ssumv)

    def zb(i, _):
        r = i // 8
        k = i % 8
        msgb[r, pl.ds(k * 16, 16)] = jnp.zeros((16,), jnp.float32)
        return 0
    lax.fori_loop(0, P2CH * 8, zb, 0, unroll=8)
    for j in range(STRIPE // P2CH):
        pltpu.sync_copy(msgb, gsh.at[pl.ds(sid * STRIPE + j * P2CH, P2CH)])
    plsc.subcore_barrier()

    iota = lax.iota(jnp.int32, 16)

    def _issue(c, b):
        pltpu.async_copy(fs_hbm.at[srcv.at[pl.ds(c * P2CH, P2CH)]], fsb[b], sem[b])

    def _wait(c, b):
        pltpu.make_async_copy(fs_hbm.at[srcv.at[pl.ds(c * P2CH, P2CH)]], fsb[b], sem[b]).wait()

    def superchunk(s, _):
        sbase = base + s * SB
        pltpu.sync_copy(src_hbm.at[pl.ds(sbase, SB)], srcv)
        pltpu.sync_copy(dst_hbm.at[pl.ds(sbase, SB)], dstv)
        pltpu.sync_copy(r_hbm.at[pl.ds(sbase, SB)], rv)
        pltpu.sync_copy(ex_hbm.at[pl.ds(sbase, SB)], exv)
        _issue(0, 0)
        _issue(1, 1)

        def chunk2(cc, _):
            for b in range(2):
                c = cc * 2 + b
                _wait(c, b)

                def group(g, _):
                    off = c * P2CH + g * 16
                    dst16 = dstv[pl.ds(off, 16)]
                    ex16 = exv[pl.ds(off, 16)]
                    sv = plsc.load_gather(ssumv, [dst16])
                    a16 = ex16 / (sv + 1e-9)
                    av[pl.ds(off, 16)] = a16
                    r16 = rv[pl.ds(off, 16)]
                    res = [r16[j] for j in range(16)]
                    for j4 in range(0, 16, 4):
                        es = [g * 16 + j4 + u for u in range(4)]
                        saccs = [jnp.zeros((16,), jnp.float32) for _ in range(4)]
                        for k in range(D // 16):
                            ks = pl.ds(k * 16, 16)
                            vs = [fsb[b][e, ks] for e in es]
                            rl = [relv[res[j4 + u], ks] for u in range(4)]
                            t = [vs[u] * rl[u] for u in range(4)]
                            t = [t[u] * a16[j4 + u] for u in range(4)]
                            p = [_pexp4(t) for t in t]
                            for u in range(4):
                                msgb[es[u], ks] = p[u]
                            saccs = [saccs[u] + p[u] for u in range(4)]
                        for u in range(4):
                            plsc.store_scatter(tbuf, [iota * 16 + j4 + u], saccs[u])
                    s16 = tbuf[pl.ds(0, 16)]
                    for i in range(1, 16):
                        s16 = s16 + tbuf[pl.ds(i * 16, 16)]
                    inv16 = 1.0 / s16
                    for k in range(D // 16):
                        for j in range(16):
                            e = g * 16 + j
                            msgb[e, pl.ds(k * 16, 16)] = msgb[e, pl.ds(k * 16, 16)] * inv16[j]
                    pltpu.sync_copy(msgb.at[pl.ds(g * 16, 16)], gsh.at[dst16], add=True)
                    return 0

                lax.fori_loop(0, GROUPS2, group, 0)

                @pl.when(c < SCHUNK - 2)
                def _():
                    _issue(c + 2, b)
            return 0

        lax.fori_loop(0, SCHUNK // 2, chunk2, 0)
        pltpu.sync_copy(av, a_hbm.at[pl.ds(sbase, SB)])
        return 0

    lax.fori_loop(0, NSB, superchunk, 0)
    plsc.subcore_barrier()
    for j in range(STRIPE // P2CH):
        r0 = sid * STRIPE + j * P2CH
        pltpu.sync_copy(gsh.at[pl.ds(r0, P2CH)],
                        g_hbm.at[pl.ds(cid * NPAD + r0, P2CH)])


def _sc_pass2(fs, srcp, dstp, rp, ex, ssum_tot, rel):
    f = pl.kernel(
        _sc_pass2_body,
        out_type=[jax.ShapeDtypeStruct((EPAD,), jnp.float32),
                  jax.ShapeDtypeStruct((NC * NPAD, D), jnp.float32)],
        mesh=_mesh,
        scratch_types=[
            pltpu.VMEM((SB,), jnp.int32),
            pltpu.VMEM((SB,), jnp.int32),
            pltpu.VMEM((SB,), jnp.int32),
            pltpu.VMEM((SB,), jnp.float32),
            pltpu.VMEM((SB,), jnp.float32),
            pltpu.VMEM((16, D), jnp.float32),
            pltpu.VMEM((NPAD,), jnp.float32),
            pltpu.VMEM((256,), jnp.float32),
            pltpu.VMEM((P2CH, D), jnp.float32),
            pltpu.VMEM((P2CH, D), jnp.float32),
            pltpu.VMEM((P2CH, D), jnp.float32),
            pltpu.SemaphoreType.DMA,
            pltpu.SemaphoreType.DMA,
            pltpu.VMEM_SHARED((NPAD, D), jnp.float32),
        ],
        compiler_params=pltpu.CompilerParams(needs_layout_passes=False),
        interpret=_IPRET,
    )
    return f(fs, srcp, dstp, rp, ex, ssum_tot, rel)


def _layer(x, srcp, dstp, rp, rel, ws, bs, wd, bd, attn):
    fs, fd = _tc_proj(x, ws, bs, wd, bd)
    ex, ssum = _sc_pass1(fs, fd, srcp, dstp, attn)
    ssum_tot = _tc_rsum(ssum).reshape(NPAD)
    a, gflat = _sc_pass2(fs, srcp, dstp, rp, ex, ssum_tot, rel)
    g0 = gflat[:N]
    g1 = gflat[NPAD:NPAD + N]
    return a, g0, g1


@jax.jit
def kernel(x, edge_index, r_type, emb_rel, W1_0, b1_0, W2s_0, b2s_0, W2d_0,
           b2d_0, attn_0, W1_1, b1_1, W2s_1, b2s_1, W2d_1, b2d_1, attn_1):
    src = edge_index[0].astype(jnp.int32)
    dst = edge_index[1].astype(jnp.int32)
    rt = r_type.astype(jnp.int32)
    npad = EPAD - E
    srcp = jnp.concatenate([src, jnp.zeros((npad,), jnp.int32)])
    dstp = jnp.concatenate([dst, jnp.full((npad,), N, jnp.int32)])
    rp = jnp.concatenate([rt, jnp.zeros((npad,), jnp.int32)])

    a1, g0, g1 = _layer(x, srcp, dstp, rp, emb_rel,
                        W2s_0, b2s_0, W2d_0, b2d_0, attn_0.reshape(D))
    x1, fs2, fd2 = _tc_mid(x, g0, g1, W1_0, b1_0, W2s_1, b2s_1, W2d_1, b2d_1)

    ex2, ssum2 = _sc_pass1(fs2, fd2, srcp, dstp, attn_1.reshape(D))
    ssum2_tot = _tc_rsum(ssum2).reshape(NPAD)
    _, gflat2 = _sc_pass2(fs2, srcp, dstp, rp, ex2, ssum2_tot, emb_rel)
    x2, gnb = _tc_fin(x1, gflat2[:N], gflat2[NPAD:NPAD + N], W1_1, b1_1)

    emb = jnp.concatenate([x1, x2], axis=0)
    attentions = a1[:E].reshape(E, 1)
    return (x2, emb, gnb, attentions)
